# baseline (speedup 1.0000x reference)
import functools

import jax
import jax.numpy as jnp
from jax import lax
from jax.experimental import pallas as pl
from jax.experimental.pallas import tpu as pltpu
from jax.experimental.pallas import tpu_sc as plsc

_LANES = 16


@functools.cache
def _make_sc_kernel(B, D, NB, bpw, chunk, nbuf):
    n_chunks = bpw // chunk
    mesh = plsc.VectorSubcoreMesh(core_axis_name="c", subcore_axis_name="s")

    @functools.partial(
        pl.kernel,
        out_type=jax.ShapeDtypeStruct((B, D), jnp.float32),
        mesh=mesh,
        compiler_params=pltpu.CompilerParams(needs_layout_passes=False),
        scratch_types=[
            pltpu.VMEM((NB,), jnp.float32),
            pltpu.VMEM((bpw,), jnp.float32),
            pltpu.VMEM((bpw,), jnp.int32),
            pltpu.VMEM((nbuf, chunk, D), jnp.float32),
        ] + [pltpu.SemaphoreType.DMA] * (2 * nbuf),
    )
    def k(clip_hbm, bnd_hbm, table_hbm, out_hbm,
          bnd_v, clip_v, idx_v, rows_v, *sems):
        nc = 2
        wid = lax.axis_index("s") * nc + lax.axis_index("c")
        base = wid * bpw
        gsem = sems[:nbuf]
        wsem = sems[nbuf:]

        pltpu.sync_copy(bnd_hbm, bnd_v)
        pltpu.sync_copy(clip_hbm.at[pl.ds(base, bpw)], clip_v)

        def bucketize(i, carry):
            x = clip_v[pl.ds(i * _LANES, _LANES)]
            lo = jnp.zeros((_LANES,), jnp.int32)
            for bit in (128, 64, 32, 16, 8, 4, 2, 1):
                probe = lo + bit
                vals = plsc.load_gather(bnd_v, [probe - 1])
                lo = jnp.where(vals <= x, probe, lo)
            idx_v[pl.ds(i * _LANES, _LANES)] = lo
            return carry

        lax.fori_loop(0, bpw // _LANES, bucketize, 0)

        def gather_desc(c):
            buf = c % nbuf
            return pltpu.make_async_copy(
                table_hbm.at[idx_v.at[pl.ds(c * chunk, chunk)]],
                rows_v.at[buf], gsem[buf])

        def write_desc(c):
            buf = c % nbuf
            return pltpu.make_async_copy(
                rows_v.at[buf], out_hbm.at[pl.ds(base + c * chunk, chunk)],
                wsem[buf])

        for c in range(nbuf - 1):
            gather_desc(c).start()
        for c in range(n_chunks):
            gather_desc(c).wait()
            write_desc(c).start()
            nxt = c + nbuf - 1
            if nxt < n_chunks:
                if c >= 1:
                    write_desc(c - 1).wait()
                gather_desc(nxt).start()
        for c in range(max(0, n_chunks - nbuf), n_chunks):
            write_desc(c).wait()

    return k


def kernel(bsz, clip_sim, boundary, control_embedding):
    B = clip_sim.shape[0]
    D = control_embedding.shape[1]
    clip = clip_sim.reshape(B)
    bnd = jnp.concatenate([boundary, jnp.full((1,), jnp.inf, jnp.float32)])
    nw = 32
    bpw = B // nw
    k = _make_sc_kernel(B, D, bnd.shape[0], bpw, 16, 6)
    return k(clip, bnd, control_embedding)


# speedup vs baseline: 1.0174x; 1.0174x over previous
import functools

import jax
import jax.numpy as jnp
from jax import lax
from jax.experimental import pallas as pl
from jax.experimental.pallas import tpu as pltpu
from jax.experimental.pallas import tpu_sc as plsc

_LANES = 16


@functools.cache
def _make_sc_kernel(B, D, NB, bpw, chunk, nbuf):
    n_chunks = bpw // chunk
    mesh = plsc.VectorSubcoreMesh(core_axis_name="c", subcore_axis_name="s")

    @functools.partial(
        pl.kernel,
        out_type=jax.ShapeDtypeStruct((B, D), jnp.float32),
        mesh=mesh,
        compiler_params=pltpu.CompilerParams(needs_layout_passes=False),
        scratch_types=[
            pltpu.VMEM((NB,), jnp.float32),
            pltpu.VMEM((bpw,), jnp.float32),
            pltpu.VMEM((bpw,), jnp.int32),
            pltpu.VMEM((nbuf, chunk, D), jnp.float32),
        ] + [pltpu.SemaphoreType.DMA] * (2 * nbuf),
    )
    def k(clip_hbm, bnd_hbm, table_hbm, out_hbm,
          bnd_v, clip_v, idx_v, rows_v, *sems):
        nc = 2
        wid = lax.axis_index("s") * nc + lax.axis_index("c")
        base = wid * bpw
        gsem = sems[:nbuf]
        wsem = sems[nbuf:]

        pltpu.sync_copy(bnd_hbm, bnd_v)
        pltpu.sync_copy(clip_hbm.at[pl.ds(base, bpw)], clip_v)

        def bucketize(i, carry):
            x = clip_v[pl.ds(i * _LANES, _LANES)]
            lo = jnp.zeros((_LANES,), jnp.int32)
            for bit in (128, 64, 32, 16, 8, 4, 2, 1):
                probe = lo + bit
                vals = plsc.load_gather(bnd_v, [probe - 1])
                lo = jnp.where(vals <= x, probe, lo)
            idx_v[pl.ds(i * _LANES, _LANES)] = lo
            return carry

        lax.fori_loop(0, bpw // _LANES, bucketize, 0)

        def gather_desc(c):
            buf = c % nbuf
            return pltpu.make_async_copy(
                table_hbm.at[idx_v.at[pl.ds(c * chunk, chunk)]],
                rows_v.at[buf], gsem[buf])

        def write_desc(c):
            buf = c % nbuf
            return pltpu.make_async_copy(
                rows_v.at[buf], out_hbm.at[pl.ds(base + c * chunk, chunk)],
                wsem[buf])

        for c in range(nbuf - 1):
            gather_desc(c).start()
        for c in range(n_chunks):
            gather_desc(c).wait()
            write_desc(c).start()
            nxt = c + nbuf - 1
            if nxt < n_chunks:
                if c >= 1:
                    write_desc(c - 1).wait()
                gather_desc(nxt).start()
        for c in range(max(0, n_chunks - nbuf), n_chunks):
            write_desc(c).wait()

    return k


def kernel(bsz, clip_sim, boundary, control_embedding):
    B = clip_sim.shape[0]
    D = control_embedding.shape[1]
    clip = clip_sim.reshape(B)
    bnd = jnp.concatenate([boundary, jnp.full((1,), jnp.inf, jnp.float32)])
    nw = 32
    bpw = B // nw
    k = _make_sc_kernel(B, D, bnd.shape[0], bpw, 32, 2)
    return k(clip, bnd, control_embedding)
